# baseline (speedup 1.0000x reference)
import functools

import jax
import jax.numpy as jnp
from jax import lax
from jax.experimental import pallas as pl
from jax.experimental.pallas import tpu as pltpu
from jax.experimental.pallas import tpu_sc as plsc

NC = 2
NS = 16
NW = NC * NS

CHUNK = 128


NBUF = 5
LA = 2


def _body(nchunks, x_hbm, table_hbm, out_hbm, idx_v, rows, gsems, wsems):
  wid = lax.axis_index("s") * NC + lax.axis_index("c")
  base = wid * nchunks
  pltpu.sync_copy(x_hbm.at[wid], idx_v)

  def gather_start(chunk, slot):
    pltpu.async_copy(table_hbm.at[idx_v.at[chunk]], rows[slot], gsems[slot])

  def gather_wait(chunk, slot):
    pltpu.make_async_copy(
        table_hbm.at[idx_v.at[chunk]], rows[slot], gsems[slot]).wait()

  def write_start(chunk, slot):
    pltpu.async_copy(rows[slot], out_hbm.at[base + chunk], wsems[slot])

  def write_wait(slot):
    pltpu.make_async_copy(rows[slot], out_hbm.at[base], wsems[slot]).wait()

  for c in range(LA):
    gather_start(c, c % NBUF)

  def step(o, carry):
    for b in range(NBUF):
      j = o * NBUF + b
      gslot = (b + LA) % NBUF

      @pl.when(j + LA < nchunks)
      def _():
        @pl.when(j + LA >= NBUF)
        def _():
          write_wait(gslot)
        gather_start(j + LA, gslot)

      gather_wait(j, b)
      write_start(j, b)
    return carry

  lax.fori_loop(0, nchunks // NBUF, step, 0, unroll=False)

  for b in range(NBUF):
    write_wait(b)


def kernel(x, table):
  B, S = x.shape
  V, D = table.shape
  n = B * S
  assert n % (NW * CHUNK) == 0 and D == CHUNK
  nchunks = n // (NW * CHUNK)
  assert nchunks % NBUF == 0

  x2 = x.reshape(NW, nchunks, CHUNK).astype(jnp.int32)

  mesh = plsc.VectorSubcoreMesh(core_axis_name="c", subcore_axis_name="s")
  k = pl.kernel(
      functools.partial(_body, nchunks),
      out_type=jax.ShapeDtypeStruct((n // CHUNK, CHUNK, D), jnp.float32),
      mesh=mesh,
      scratch_types=[
          pltpu.VMEM((nchunks, CHUNK), jnp.int32),
          [pltpu.VMEM((CHUNK, D), jnp.float32) for _ in range(NBUF)],
          [pltpu.SemaphoreType.DMA for _ in range(NBUF)],
          [pltpu.SemaphoreType.DMA for _ in range(NBUF)],
      ],
  )
  out = k(x2, table)
  return out.reshape(B, S, D)


# speedup vs baseline: 1.7806x; 1.7806x over previous
import functools

import jax
import jax.numpy as jnp
from jax import lax
from jax.experimental import pallas as pl
from jax.experimental.pallas import tpu as pltpu
from jax.experimental.pallas import tpu_sc as plsc

NC = 2
NS = 16
NW = NC * NS

NBUF = 8
LA = 3


def _body(bpw, x_hbm, table_hbm, out_hbm, idx_v, rows, gsems, wsems):
  wid = lax.axis_index("s") * NC + lax.axis_index("c")
  base = wid * bpw
  pltpu.sync_copy(x_hbm.at[pl.ds(base, bpw)], idx_v)

  def gather_start(b, slot):
    pltpu.async_copy(table_hbm.at[idx_v.at[b]], rows[slot], gsems[slot])

  def gather_wait(b, slot):
    pltpu.make_async_copy(
        table_hbm.at[idx_v.at[b]], rows[slot], gsems[slot]).wait()

  def write_start(b, slot):
    pltpu.async_copy(rows[slot], out_hbm.at[base + b], wsems[slot])

  def write_wait(slot):
    pltpu.make_async_copy(rows[slot], out_hbm.at[base], wsems[slot]).wait()

  for c in range(LA):
    gather_start(c, c % NBUF)

  def step(o, carry):
    for sl in range(NBUF):
      b = o * NBUF + sl
      gslot = (sl + LA) % NBUF

      @pl.when(b + LA < bpw)
      def _():
        @pl.when(b + LA >= NBUF)
        def _():
          write_wait(gslot)
        gather_start(b + LA, gslot)

      gather_wait(b, sl)
      write_start(b, sl)
    return carry

  lax.fori_loop(0, bpw // NBUF, step, 0, unroll=False)

  for sl in range(NBUF):
    write_wait(sl)


def kernel(x, table):
  B, S = x.shape
  V, D = table.shape
  assert B % NW == 0
  bpw = B // NW
  assert bpw % NBUF == 0 and (B // NW) % 8 == 0

  mesh = plsc.VectorSubcoreMesh(core_axis_name="c", subcore_axis_name="s")
  k = pl.kernel(
      functools.partial(_body, bpw),
      out_type=jax.ShapeDtypeStruct((B, S, D), jnp.float32),
      mesh=mesh,
      scratch_types=[
          pltpu.VMEM((bpw, S), jnp.int32),
          [pltpu.VMEM((S, D), jnp.float32) for _ in range(NBUF)],
          [pltpu.SemaphoreType.DMA for _ in range(NBUF)],
          [pltpu.SemaphoreType.DMA for _ in range(NBUF)],
      ],
  )
  return k(x.astype(jnp.int32), table)


# speedup vs baseline: 1.7838x; 1.0018x over previous
import functools

import jax
import jax.numpy as jnp
from jax import lax
from jax.experimental import pallas as pl
from jax.experimental.pallas import tpu as pltpu
from jax.experimental.pallas import tpu_sc as plsc

NC = 2
NS = 16
NW = NC * NS

NBUF = 8
LA = 3


def _body(bpw, x_hbm, table_hbm, out_hbm, idx_v, rows, gsems, wsems):
  wid = lax.axis_index("s") * NC + lax.axis_index("c")
  base = wid * bpw
  pltpu.sync_copy(x_hbm.at[pl.ds(base, bpw)], idx_v)

  def gather_start(b, slot):
    pltpu.async_copy(table_hbm.at[idx_v.at[b]], rows[slot], gsems[slot])

  def gather_wait(b, slot):
    pltpu.make_async_copy(
        table_hbm.at[idx_v.at[b]], rows[slot], gsems[slot]).wait()

  def write_start(b, slot):
    pltpu.async_copy(rows[slot], out_hbm.at[base + b], wsems[slot])

  def write_wait(slot):
    pltpu.make_async_copy(rows[slot], out_hbm.at[base], wsems[slot]).wait()

  for c in range(LA):
    gather_start(c, c % NBUF)

  def step(o, carry):
    for sl in range(NBUF):
      b = o * NBUF + sl
      gslot = (sl + LA) % NBUF

      @pl.when(b + LA < bpw)
      def _():
        @pl.when(b + LA >= NBUF)
        def _():
          write_wait(gslot)
        gather_start(b + LA, gslot)

      gather_wait(b, sl)
      write_start(b, sl)
    return carry

  lax.fori_loop(0, bpw // NBUF, step, 0, unroll=False)

  for sl in range(NBUF):
    write_wait(sl)


def kernel(x, table):
  B, S = x.shape
  V, D = table.shape
  assert B % NW == 0
  bpw = B // NW
  assert bpw % NBUF == 0 and (B // NW) % 8 == 0

  mesh = plsc.VectorSubcoreMesh(core_axis_name="c", subcore_axis_name="s")
  k = pl.kernel(
      functools.partial(_body, bpw),
      out_type=jax.ShapeDtypeStruct((B, S, D), jnp.float32),
      mesh=mesh,
      compiler_params=pltpu.CompilerParams(use_tc_tiling_on_sc=True),
      scratch_types=[
          pltpu.VMEM((bpw, S), jnp.int32),
          [pltpu.VMEM((S, D), jnp.float32) for _ in range(NBUF)],
          [pltpu.SemaphoreType.DMA for _ in range(NBUF)],
          [pltpu.SemaphoreType.DMA for _ in range(NBUF)],
      ],
  )
  return k(x.astype(jnp.int32), table)


# speedup vs baseline: 3.1082x; 1.7425x over previous
import functools

import jax
import jax.numpy as jnp
from jax import lax
from jax.experimental import pallas as pl
from jax.experimental.pallas import tpu as pltpu
from jax.experimental.pallas import tpu_sc as plsc

NC = 2
NS = 16
NW = NC * NS

CHUNK = 128
NBUF = 5
LA = 3


def _body(nchunks, x_hbm, table_hbm, out_hbm, idx_v, rows, gsems, wsems):
  wid = lax.axis_index("s") * NC + lax.axis_index("c")
  base = wid * nchunks
  pltpu.sync_copy(x_hbm.at[wid], idx_v)

  def gather_start(c, slot):
    pltpu.async_copy(table_hbm.at[idx_v.at[c]], rows[slot], gsems[slot])

  def gather_wait(c, slot):
    pltpu.make_async_copy(
        table_hbm.at[idx_v.at[c]], rows[slot], gsems[slot]).wait()

  def write_start(c, slot):
    pltpu.async_copy(rows[slot], out_hbm.at[base + c], wsems[slot])

  def write_wait(slot):
    pltpu.make_async_copy(rows[slot], out_hbm.at[base], wsems[slot]).wait()

  for c in range(LA):
    gather_start(c, c % NBUF)

  def step(o, carry):
    for sl in range(NBUF):
      c = o * NBUF + sl
      gslot = (sl + LA) % NBUF

      @pl.when(c + LA < nchunks)
      def _():
        @pl.when(c + LA >= NBUF)
        def _():
          write_wait(gslot)
        gather_start(c + LA, gslot)

      gather_wait(c, sl)
      write_start(c, sl)
    return carry

  lax.fori_loop(0, nchunks // NBUF, step, 0, unroll=False)

  for sl in range(NBUF):
    write_wait(sl)


def kernel(x, table):
  B, S = x.shape
  V, D = table.shape
  n = B * S
  assert n % (NW * CHUNK) == 0 and D == CHUNK and B % CHUNK == 0
  nchunks = n // (NW * CHUNK)
  assert nchunks % NBUF == 0

  xs = x.T.reshape(NW, nchunks, CHUNK).astype(jnp.int32)

  mesh = plsc.VectorSubcoreMesh(core_axis_name="c", subcore_axis_name="s")
  k = pl.kernel(
      functools.partial(_body, nchunks),
      out_type=jax.ShapeDtypeStruct((n // CHUNK, CHUNK, D), jnp.float32),
      mesh=mesh,
      scratch_types=[
          pltpu.VMEM((nchunks, CHUNK), jnp.int32),
          [pltpu.VMEM((CHUNK, D), jnp.float32) for _ in range(NBUF)],
          [pltpu.SemaphoreType.DMA for _ in range(NBUF)],
          [pltpu.SemaphoreType.DMA for _ in range(NBUF)],
      ],
  )
  out = k(xs, table)
  return out.reshape(S, B, D).transpose(1, 0, 2)
